# baseline (speedup 1.0000x reference)
import functools

import jax
import jax.numpy as jnp
from jax import lax
from jax.experimental import pallas as pl
from jax.experimental.pallas import tpu as pltpu
from jax.experimental.pallas import tpu_sc as plsc

N = 10000
E = 320000
F = 128
H = 16
NC, NS, L = 2, 16, 16
NW = NC * NS
CHUNK = 128
NBUF = 8
CPW = -(-E // (NW * CHUNK * NBUF)) * NBUF
EPAD = NW * CPW * CHUNK
NPAD = ((N + 1 + NS * 8 - 1) // (NS * 8)) * (NS * 8)
RPS = NPAD // NS

_mesh = plsc.VectorSubcoreMesh(core_axis_name="c", subcore_axis_name="s")
_sc_params = pltpu.CompilerParams(use_tc_tiling_on_sc=False)


def _sc_degree(dst_r):
    @functools.partial(
        pl.kernel,
        out_type=jax.ShapeDtypeStruct((NC, NPAD, L), jnp.float32),
        mesh=_mesh,
        compiler_params=_sc_params,
        scratch_types=[
            pltpu.VMEM((CPW, CHUNK), jnp.int32),
            pltpu.VMEM((CHUNK, L), jnp.float32),
            pltpu.VMEM((RPS, L), jnp.float32),
            pltpu.VMEM_SHARED((NPAD, L), jnp.float32),
            pltpu.SemaphoreType.DMA,
        ],
    )
    def k(dst_hbm, out_hbm, dst_v, ones_v, stage_v, acc_sh, sem):
        cid = lax.axis_index("c")
        sid = lax.axis_index("s")
        wid = sid * NC + cid

        @pl.loop(0, CHUNK)
        def _(i):
            ones_v[i, :] = jnp.ones((L,), jnp.float32)

        @pl.loop(0, RPS)
        def _(i):
            stage_v[i, :] = jnp.zeros((L,), jnp.float32)

        pltpu.sync_copy(stage_v, acc_sh.at[pl.ds(sid * RPS, RPS)])
        plsc.subcore_barrier()

        pltpu.sync_copy(dst_hbm.at[wid], dst_v)

        @pl.loop(0, CPW)
        def _(j):
            pltpu.async_copy(ones_v, acc_sh.at[dst_v.at[j]], sem, add=True)

        @pl.loop(0, CPW)
        def _(j):
            pltpu.make_async_copy(ones_v, acc_sh.at[dst_v.at[j]], sem).wait()

        plsc.subcore_barrier()
        pltpu.sync_copy(
            acc_sh.at[pl.ds(sid * RPS, RPS)],
            out_hbm.at[cid].at[pl.ds(sid * RPS, RPS)],
        )

    return k(dst_r)


def _sc_aggregate(g, src_r, dst_r):
    @functools.partial(
        pl.kernel,
        out_type=jax.ShapeDtypeStruct((NC, NPAD, L), jnp.float32),
        mesh=_mesh,
        compiler_params=_sc_params,
        scratch_types=[
            pltpu.VMEM((CPW, CHUNK), jnp.int32),
            pltpu.VMEM((CPW, CHUNK), jnp.int32),
            pltpu.VMEM((NBUF, CHUNK, L), jnp.float32),
            pltpu.VMEM((RPS, L), jnp.float32),
            pltpu.VMEM_SHARED((NPAD, L), jnp.float32),
            pltpu.SemaphoreType.DMA((NBUF,)),
            pltpu.SemaphoreType.DMA((NBUF,)),
        ],
    )
    def k(g_hbm, src_hbm, dst_hbm, out_hbm, src_v, dst_v, rows_v, stage_v,
          acc_sh, gsem, ssem):
        cid = lax.axis_index("c")
        sid = lax.axis_index("s")
        wid = sid * NC + cid

        @pl.loop(0, RPS)
        def _(i):
            stage_v[i, :] = jnp.zeros((L,), jnp.float32)

        pltpu.sync_copy(stage_v, acc_sh.at[pl.ds(sid * RPS, RPS)])
        plsc.subcore_barrier()

        pltpu.sync_copy(src_hbm.at[wid], src_v)
        pltpu.sync_copy(dst_hbm.at[wid], dst_v)

        for b in range(NBUF):
            pltpu.async_copy(g_hbm.at[src_v.at[b]], rows_v.at[b], gsem.at[b])

        @pl.loop(0, CPW - NBUF, step=NBUF)
        def _(j):
            for b in range(NBUF):
                pltpu.make_async_copy(
                    g_hbm.at[src_v.at[j + b]], rows_v.at[b], gsem.at[b]
                ).wait()
                pltpu.async_copy(
                    rows_v.at[b], acc_sh.at[dst_v.at[j + b]], ssem.at[b],
                    add=True)
                pltpu.make_async_copy(
                    rows_v.at[b], acc_sh.at[dst_v.at[j + b]], ssem.at[b]
                ).wait()
                pltpu.async_copy(
                    g_hbm.at[src_v.at[j + NBUF + b]], rows_v.at[b],
                    gsem.at[b])

        for b in range(NBUF):
            jb = CPW - NBUF + b
            pltpu.make_async_copy(
                g_hbm.at[src_v.at[jb]], rows_v.at[b], gsem.at[b]).wait()
            pltpu.sync_copy(rows_v.at[b], acc_sh.at[dst_v.at[jb]], add=True)

        plsc.subcore_barrier()
        pltpu.sync_copy(
            acc_sh.at[pl.ds(sid * RPS, RPS)],
            out_hbm.at[cid].at[pl.ds(sid * RPS, RPS)],
        )

    return k(g, src_r, dst_r)


def _tc_matmul1(x, W1):
    def body(x_ref, w_ref, o_ref):
        o_ref[...] = jnp.dot(x_ref[...], w_ref[...],
                             preferred_element_type=jnp.float32)

    return pl.pallas_call(
        body, out_shape=jax.ShapeDtypeStruct((N, H), jnp.float32))(x, W1)


def _tc_prep(deg2, h1):
    def body(d_ref, h_ref, dinv_ref, g_ref):
        deg = d_ref[0, :N, 0:1] + d_ref[1, :N, 0:1] + 1.0
        dinv = lax.rsqrt(deg)
        dinv_ref[...] = dinv
        g_ref[...] = h_ref[...] * dinv

    return pl.pallas_call(
        body,
        out_shape=(jax.ShapeDtypeStruct((N, 1), jnp.float32),
                   jax.ShapeDtypeStruct((N, H), jnp.float32)),
    )(deg2, h1)


def _tc_mid(S1, g1, dinv, b1, W2):
    def body(s_ref, g_ref, dinv_ref, b_ref, w_ref, o_ref):
        s = s_ref[0, :N, :] + s_ref[1, :N, :] + g_ref[...]
        out1 = jnp.maximum(dinv_ref[...] * s + b_ref[...], 0.0)
        h2 = jnp.dot(out1, w_ref[...], preferred_element_type=jnp.float32)
        o_ref[...] = dinv_ref[...] * h2

    return pl.pallas_call(
        body, out_shape=jax.ShapeDtypeStruct((N, H), jnp.float32)
    )(S1, g1, dinv, b1, W2)


def _tc_final(S2, g2, dinv, b2):
    def body(s_ref, g_ref, dinv_ref, b_ref, o_ref):
        s = s_ref[0, :N, :] + s_ref[1, :N, :] + g_ref[...]
        o = dinv_ref[...] * s + b_ref[...]
        m = jnp.max(o, axis=1, keepdims=True)
        e = jnp.exp(o - m)
        lse = jnp.log(jnp.sum(e, axis=1, keepdims=True)) + m
        o_ref[...] = o - lse

    return pl.pallas_call(
        body, out_shape=jax.ShapeDtypeStruct((N, H), jnp.float32)
    )(S2, g2, dinv, b2)


def kernel(x, edge_index, W1, b1, W2, b2):
    ei = edge_index.astype(jnp.int32)
    pad = EPAD - E
    src = jnp.concatenate([ei[0], jnp.zeros((pad,), jnp.int32)])
    dst = jnp.concatenate([ei[1], jnp.full((pad,), N, jnp.int32)])
    src_r = src.reshape(NW, CPW, CHUNK)
    dst_r = dst.reshape(NW, CPW, CHUNK)
    b1r = b1.reshape(1, H)
    b2r = b2.reshape(1, H)

    deg2 = _sc_degree(dst_r)
    h1 = _tc_matmul1(x, W1)
    dinv, g1 = _tc_prep(deg2, h1)
    S1 = _sc_aggregate(g1, src_r, dst_r)
    g2 = _tc_mid(S1, g1, dinv, b1r, W2)
    S2 = _sc_aggregate(g2, src_r, dst_r)
    return _tc_final(S2, g2, dinv, b2r)


# speedup vs baseline: 1.4649x; 1.4649x over previous
import functools

import jax
import jax.numpy as jnp
from jax import lax
from jax.experimental import pallas as pl
from jax.experimental.pallas import tpu as pltpu
from jax.experimental.pallas import tpu_sc as plsc

N = 10000
E = 320000
F = 128
H = 16
NC, NS, L = 2, 16, 16
NW = NC * NS
CHUNK = 128
NBUF = 4
CPW = -(-E // (NW * CHUNK * NBUF)) * NBUF
EPAD = NW * CPW * CHUNK
NPAD = ((N + 1 + NS * 8 - 1) // (NS * 8)) * (NS * 8)
RPS = NPAD // NS

_mesh = plsc.VectorSubcoreMesh(core_axis_name="c", subcore_axis_name="s")
_sc_params = pltpu.CompilerParams(use_tc_tiling_on_sc=False)


def _sc_degree(dst_r):
    @functools.partial(
        pl.kernel,
        out_type=jax.ShapeDtypeStruct((NC, NPAD, L), jnp.float32),
        mesh=_mesh,
        compiler_params=_sc_params,
        scratch_types=[
            pltpu.VMEM((CPW, CHUNK), jnp.int32),
            pltpu.VMEM((CHUNK, L), jnp.float32),
            pltpu.VMEM((RPS, L), jnp.float32),
            pltpu.VMEM_SHARED((NPAD, L), jnp.float32),
            pltpu.SemaphoreType.DMA,
        ],
    )
    def k(dst_hbm, out_hbm, dst_v, ones_v, stage_v, acc_sh, sem):
        cid = lax.axis_index("c")
        sid = lax.axis_index("s")
        wid = sid * NC + cid

        @pl.loop(0, CHUNK)
        def _(i):
            ones_v[i, :] = jnp.ones((L,), jnp.float32)

        @pl.loop(0, RPS)
        def _(i):
            stage_v[i, :] = jnp.zeros((L,), jnp.float32)

        pltpu.sync_copy(stage_v, acc_sh.at[pl.ds(sid * RPS, RPS)])
        plsc.subcore_barrier()

        pltpu.sync_copy(dst_hbm.at[wid], dst_v)

        @pl.loop(0, CPW)
        def _(j):
            pltpu.async_copy(ones_v, acc_sh.at[dst_v.at[j]], sem, add=True)

        @pl.loop(0, CPW)
        def _(j):
            pltpu.make_async_copy(ones_v, acc_sh.at[dst_v.at[j]], sem).wait()

        plsc.subcore_barrier()
        pltpu.sync_copy(
            acc_sh.at[pl.ds(sid * RPS, RPS)],
            out_hbm.at[cid].at[pl.ds(sid * RPS, RPS)],
        )

    return k(dst_r)


def _sc_aggregate(g, src_r, dst_r):
    @functools.partial(
        pl.kernel,
        out_type=jax.ShapeDtypeStruct((NC, NPAD, L), jnp.float32),
        mesh=_mesh,
        compiler_params=_sc_params,
        scratch_types=[
            pltpu.VMEM((CPW, CHUNK), jnp.int32),
            pltpu.VMEM((CPW, CHUNK), jnp.int32),
            pltpu.VMEM((NBUF, CHUNK, L), jnp.float32),
            pltpu.VMEM((RPS, L), jnp.float32),
            pltpu.VMEM_SHARED((NPAD, L), jnp.float32),
            pltpu.VMEM_SHARED((NPAD, L), jnp.float32),
            pltpu.SemaphoreType.DMA((NBUF,)),
            pltpu.SemaphoreType.DMA((NBUF,)),
            pltpu.SemaphoreType.DMA,
        ],
    )
    def k(g_hbm, src_hbm, dst_hbm, out_hbm, src_v, dst_v, rows_v, stage_v,
          acc_sh, g_sh, gsem, ssem, psem):
        cid = lax.axis_index("c")
        sid = lax.axis_index("s")
        wid = sid * NC + cid

        sl = pl.ds(sid * RPS, RPS)
        pltpu.async_copy(g_hbm.at[sl], g_sh.at[sl], psem)

        @pl.loop(0, RPS)
        def _(i):
            stage_v[i, :] = jnp.zeros((L,), jnp.float32)

        pltpu.sync_copy(stage_v, acc_sh.at[sl])
        pltpu.sync_copy(src_hbm.at[wid], src_v)
        pltpu.sync_copy(dst_hbm.at[wid], dst_v)
        pltpu.make_async_copy(g_hbm.at[sl], g_sh.at[sl], psem).wait()
        plsc.subcore_barrier()

        for b in range(NBUF):
            pltpu.async_copy(g_sh.at[src_v.at[b]], rows_v.at[b], gsem.at[b])

        @pl.loop(0, CPW - NBUF, step=NBUF)
        def _(j):
            for b in range(NBUF):
                pltpu.make_async_copy(
                    g_sh.at[src_v.at[j + b]], rows_v.at[b], gsem.at[b]
                ).wait()
                pltpu.async_copy(
                    rows_v.at[b], acc_sh.at[dst_v.at[j + b]], ssem.at[b],
                    add=True)
                pltpu.make_async_copy(
                    rows_v.at[b], acc_sh.at[dst_v.at[j + b]], ssem.at[b]
                ).wait()
                pltpu.async_copy(
                    g_sh.at[src_v.at[j + NBUF + b]], rows_v.at[b],
                    gsem.at[b])

        for b in range(NBUF):
            jb = CPW - NBUF + b
            pltpu.make_async_copy(
                g_sh.at[src_v.at[jb]], rows_v.at[b], gsem.at[b]).wait()
            pltpu.sync_copy(rows_v.at[b], acc_sh.at[dst_v.at[jb]], add=True)

        plsc.subcore_barrier()
        pltpu.sync_copy(
            acc_sh.at[pl.ds(sid * RPS, RPS)],
            out_hbm.at[cid].at[pl.ds(sid * RPS, RPS)],
        )

    return k(g, src_r, dst_r)


def _tc_matmul1(x, W1):
    def body(x_ref, w_ref, o_ref):
        o_ref[...] = jnp.dot(x_ref[...], w_ref[...],
                             preferred_element_type=jnp.float32)

    return pl.pallas_call(
        body, out_shape=jax.ShapeDtypeStruct((NPAD, H), jnp.float32))(x, W1)


def _tc_prep(deg2, h1):
    def body(d_ref, h_ref, dinv_ref, g_ref):
        deg = d_ref[0, :, 0:1] + d_ref[1, :, 0:1] + 1.0
        dinv = lax.rsqrt(deg)
        dinv_ref[...] = dinv
        g_ref[...] = h_ref[...] * dinv

    return pl.pallas_call(
        body,
        out_shape=(jax.ShapeDtypeStruct((NPAD, 1), jnp.float32),
                   jax.ShapeDtypeStruct((NPAD, H), jnp.float32)),
    )(deg2, h1)


def _tc_mid(S1, g1, dinv, b1, W2):
    def body(s_ref, g_ref, dinv_ref, b_ref, w_ref, o_ref):
        s = s_ref[0, :, :] + s_ref[1, :, :] + g_ref[...]
        out1 = jnp.maximum(dinv_ref[...] * s + b_ref[...], 0.0)
        h2 = jnp.dot(out1, w_ref[...], preferred_element_type=jnp.float32)
        o_ref[...] = dinv_ref[...] * h2

    return pl.pallas_call(
        body, out_shape=jax.ShapeDtypeStruct((NPAD, H), jnp.float32)
    )(S1, g1, dinv, b1, W2)


def _tc_final(S2, g2, dinv, b2):
    def body(s_ref, g_ref, dinv_ref, b_ref, o_ref):
        s = s_ref[0, :N, :] + s_ref[1, :N, :] + g_ref[:N, :]
        o = dinv_ref[:N, :] * s + b_ref[...]
        m = jnp.max(o, axis=1, keepdims=True)
        e = jnp.exp(o - m)
        lse = jnp.log(jnp.sum(e, axis=1, keepdims=True)) + m
        o_ref[...] = o - lse

    return pl.pallas_call(
        body, out_shape=jax.ShapeDtypeStruct((N, H), jnp.float32)
    )(S2, g2, dinv, b2)


def kernel(x, edge_index, W1, b1, W2, b2):
    x_pad = jnp.pad(x, ((0, NPAD - N), (0, 0)))
    ei = edge_index.astype(jnp.int32)
    pad = EPAD - E
    src = jnp.concatenate([ei[0], jnp.zeros((pad,), jnp.int32)])
    dst = jnp.concatenate([ei[1], jnp.full((pad,), N, jnp.int32)])
    src_r = src.reshape(NW, CPW, CHUNK)
    dst_r = dst.reshape(NW, CPW, CHUNK)
    b1r = b1.reshape(1, H)
    b2r = b2.reshape(1, H)

    deg2 = _sc_degree(dst_r)
    h1 = _tc_matmul1(x_pad, W1)
    dinv, g1 = _tc_prep(deg2, h1)
    S1 = _sc_aggregate(g1, src_r, dst_r)
    g2 = _tc_mid(S1, g1, dinv, b1r, W2)
    S2 = _sc_aggregate(g2, src_r, dst_r)
    return _tc_final(S2, g2, dinv, b2r)
